# baseline (speedup 1.0000x reference)
import functools

import jax
import jax.numpy as jnp
from jax import lax
from jax.experimental import pallas as pl
from jax.experimental.pallas import tpu as pltpu
from jax.experimental.pallas import tpu_sc as plsc

BATCH = 16384
EMBED = 32
WIDE = 128
PACK = WIDE // EMBED
NC, NS = 2, 16
NW = NC * NS
B_PER_W = BATCH // NW
CHUNK = 128
NCHUNK = B_PER_W // CHUNK
LANES = 16


def _sc_gather_body(uidx_hbm, iidx_hbm, utab_hbm, itab_hbm,
                    uout_hbm, iout_hbm,
                    uidx_v, iidx_v, ubase_v, ibase_v, rows_v, sem):
    wid = lax.axis_index("s") * NC + lax.axis_index("c")
    base = wid * B_PER_W
    pltpu.sync_copy(uidx_hbm.at[pl.ds(base, B_PER_W)], uidx_v)
    pltpu.sync_copy(iidx_hbm.at[pl.ds(base, B_PER_W)], iidx_v)
    for t in range(B_PER_W // LANES):
        sl = pl.ds(t * LANES, LANES)
        ubase_v[sl] = jax.lax.shift_right_logical(uidx_v[sl], 2)
        ibase_v[sl] = jax.lax.shift_right_logical(iidx_v[sl], 2)
    for tab_hbm, base_v, out_hbm in ((utab_hbm, ubase_v, uout_hbm),
                                     (itab_hbm, ibase_v, iout_hbm)):
        copies = []
        for j in range(NCHUNK):
            sl = pl.ds(j * CHUNK, CHUNK)
            copies.append(pltpu.async_copy(tab_hbm.at[base_v.at[sl]],
                                           rows_v.at[sl], sem))
        for c in copies:
            c.wait()
        pltpu.sync_copy(rows_v, out_hbm.at[pl.ds(base, B_PER_W)])


@jax.jit
def _sc_gather(user, item, user_table, item_table):
    mesh = plsc.VectorSubcoreMesh(core_axis_name="c", subcore_axis_name="s")
    k = functools.partial(
        pl.kernel,
        mesh=mesh,
        out_type=[jax.ShapeDtypeStruct((BATCH, WIDE), jnp.float32),
                  jax.ShapeDtypeStruct((BATCH, WIDE), jnp.float32)],
        scratch_types=[
            pltpu.VMEM((B_PER_W,), jnp.int32),
            pltpu.VMEM((B_PER_W,), jnp.int32),
            pltpu.VMEM((B_PER_W,), jnp.int32),
            pltpu.VMEM((B_PER_W,), jnp.int32),
            pltpu.VMEM((B_PER_W, WIDE), jnp.float32),
            pltpu.SemaphoreType.DMA,
        ],
    )(_sc_gather_body)
    return k(user, item, user_table, item_table)


def _mlp_body(ub_ref, ib_ref, uidx_ref, iidx_ref,
              w1_ref, b1_ref, w2_ref, b2_ref, o_ref):
    ub = ub_ref[...]
    ib = ib_ref[...]
    uoff = uidx_ref[...] & (PACK - 1)
    ioff = iidx_ref[...] & (PACK - 1)
    u = jnp.where(uoff == 0, ub[:, 0:EMBED], 0.0)
    i = jnp.where(ioff == 0, ib[:, 0:EMBED], 0.0)
    for k in range(1, PACK):
        u = u + jnp.where(uoff == k, ub[:, k * EMBED:(k + 1) * EMBED], 0.0)
        i = i + jnp.where(ioff == k, ib[:, k * EMBED:(k + 1) * EMBED], 0.0)
    x = jnp.dot(u, w1_ref[0:EMBED, :], preferred_element_type=jnp.float32)
    x = x + jnp.dot(i, w1_ref[EMBED:2 * EMBED, :],
                    preferred_element_type=jnp.float32)
    x = jnp.maximum(x + b1_ref[...], 0.0)
    o_ref[...] = jnp.dot(x, w2_ref[...], preferred_element_type=jnp.float32) + b2_ref[...]


@jax.jit
def _tc_mlp(u_big, i_big, user, item, W1, b1, W2, b2):
    R = 2048
    grid = (BATCH // R,)
    return pl.pallas_call(
        _mlp_body,
        grid=grid,
        in_specs=[
            pl.BlockSpec((R, WIDE), lambda r: (r, 0)),
            pl.BlockSpec((R, WIDE), lambda r: (r, 0)),
            pl.BlockSpec((R, 1), lambda r: (r, 0)),
            pl.BlockSpec((R, 1), lambda r: (r, 0)),
            pl.BlockSpec((2 * EMBED, 128), lambda r: (0, 0)),
            pl.BlockSpec((1, 128), lambda r: (0, 0)),
            pl.BlockSpec((128, 5), lambda r: (0, 0)),
            pl.BlockSpec((1, 5), lambda r: (0, 0)),
        ],
        out_specs=pl.BlockSpec((R, 5), lambda r: (r, 0)),
        out_shape=jax.ShapeDtypeStruct((BATCH, 5), jnp.float32),
    )(u_big, i_big, user.reshape(BATCH, 1), item.reshape(BATCH, 1),
      W1, b1.reshape(1, 128), W2, b2.reshape(1, 5))


def kernel(user, item, user_table, item_table, W1, b1, W2, b2):
    utab_wide = user_table.reshape(-1, WIDE)
    itab_wide = item_table.reshape(-1, WIDE)
    u_big, i_big = _sc_gather(user, item, utab_wide, itab_wide)
    return _tc_mlp(u_big, i_big, user, item, W1, b1, W2, b2)


# speedup vs baseline: 1.5352x; 1.5352x over previous
import functools

import jax
import jax.numpy as jnp
from jax import lax
from jax.experimental import pallas as pl
from jax.experimental.pallas import tpu as pltpu
from jax.experimental.pallas import tpu_sc as plsc

BATCH = 16384
EMBED = 32
NC, NS = 2, 16
NW = NC * NS
B_PER_W = BATCH // NW
LANES = 16


def _sc_gather_body(uidx_hbm, iidx_hbm, utab_hbm, itab_hbm,
                    uout_hbm, iout_hbm,
                    uidx_v, iidx_v, rows_v, sem):
    wid = lax.axis_index("s") * NC + lax.axis_index("c")
    base = wid * B_PER_W
    pltpu.sync_copy(uidx_hbm.at[pl.ds(base, B_PER_W)], uidx_v)
    pltpu.sync_copy(iidx_hbm.at[pl.ds(base, B_PER_W)], iidx_v)
    for tab_hbm, idx_v, out_hbm in ((utab_hbm, uidx_v, uout_hbm),
                                    (itab_hbm, iidx_v, iout_hbm)):
        def issue(g, _):
            v = idx_v[pl.ds(g * LANES, LANES)]
            for t in range(LANES):
                pltpu.async_copy(tab_hbm.at[pl.ds(v[t], 1)],
                                 rows_v.at[pl.ds(g * LANES + t, 1)], sem)
            return _
        lax.fori_loop(0, B_PER_W // LANES, issue, None)

        def drain(n, _):
            pltpu.make_async_copy(tab_hbm.at[pl.ds(0, 1)],
                                  rows_v.at[pl.ds(0, 1)], sem).wait()
            return _
        lax.fori_loop(0, B_PER_W, drain, None)
        pltpu.sync_copy(rows_v, out_hbm.at[pl.ds(base, B_PER_W)])


@jax.jit
def _sc_gather(user, item, user_table, item_table):
    mesh = plsc.VectorSubcoreMesh(core_axis_name="c", subcore_axis_name="s")
    k = functools.partial(
        pl.kernel,
        mesh=mesh,
        out_type=[jax.ShapeDtypeStruct((BATCH, EMBED), jnp.float32),
                  jax.ShapeDtypeStruct((BATCH, EMBED), jnp.float32)],
        scratch_types=[
            pltpu.VMEM((B_PER_W,), jnp.int32),
            pltpu.VMEM((B_PER_W,), jnp.int32),
            pltpu.VMEM((B_PER_W, EMBED), jnp.float32),
            pltpu.SemaphoreType.DMA,
        ],
    )(_sc_gather_body)
    return k(user, item, user_table, item_table)


def _mlp_body(u_ref, i_ref, w1_ref, b1_ref, w2_ref, b2_ref, o_ref):
    x = jnp.dot(u_ref[...], w1_ref[0:EMBED, :], preferred_element_type=jnp.float32)
    x = x + jnp.dot(i_ref[...], w1_ref[EMBED:2 * EMBED, :],
                    preferred_element_type=jnp.float32)
    x = jnp.maximum(x + b1_ref[...], 0.0)
    o_ref[...] = jnp.dot(x, w2_ref[...], preferred_element_type=jnp.float32) + b2_ref[...]


@jax.jit
def _tc_mlp(u_emb, i_emb, W1, b1, W2, b2):
    R = 2048
    grid = (BATCH // R,)
    return pl.pallas_call(
        _mlp_body,
        grid=grid,
        in_specs=[
            pl.BlockSpec((R, EMBED), lambda r: (r, 0)),
            pl.BlockSpec((R, EMBED), lambda r: (r, 0)),
            pl.BlockSpec((2 * EMBED, 128), lambda r: (0, 0)),
            pl.BlockSpec((1, 128), lambda r: (0, 0)),
            pl.BlockSpec((128, 5), lambda r: (0, 0)),
            pl.BlockSpec((1, 5), lambda r: (0, 0)),
        ],
        out_specs=pl.BlockSpec((R, 5), lambda r: (r, 0)),
        out_shape=jax.ShapeDtypeStruct((BATCH, 5), jnp.float32),
    )(u_emb, i_emb, W1, b1.reshape(1, 128), W2, b2.reshape(1, 5))


def kernel(user, item, user_table, item_table, W1, b1, W2, b2):
    u_emb, i_emb = _sc_gather(user, item, user_table, item_table)
    return _tc_mlp(u_emb, i_emb, W1, b1, W2, b2)
